# baseline (speedup 1.0000x reference)
import functools

import jax
import jax.numpy as jnp
from jax import lax
from jax.experimental import pallas as pl
from jax.experimental.pallas import tpu as pltpu
from jax.experimental.pallas import tpu_sc as plsc

N = 10000
NPAD = 10112
D = 128
DEGW = 128
CHUNK = 128
NCHUNK = 80
NTILES = 32
EPAD = NTILES * NCHUNK * CHUNK
NB = 1000
GRID = N // NB

_ZROWS = NPAD // 16


@functools.cache
def _sc_kernels():
    mesh = plsc.VectorSubcoreMesh(core_axis_name="c", subcore_axis_name="s")

    GB = 8
    NGRP = NCHUNK // GB

    @functools.partial(
        pl.kernel,
        out_type=jax.ShapeDtypeStruct((2, NPAD, D), jnp.float32),
        mesh=mesh,
        scratch_types=[
            pltpu.VMEM((GB, CHUNK), jnp.int32),
            pltpu.VMEM((GB, CHUNK), jnp.int32),
            pltpu.VMEM((GB, CHUNK), jnp.int32),
            pltpu.VMEM((GB, CHUNK), jnp.int32),
            pltpu.VMEM((CHUNK, D), jnp.float32),
            pltpu.VMEM((CHUNK, D), jnp.float32),
            pltpu.VMEM_SHARED((NPAD, D), jnp.float32),
            pltpu.SemaphoreType.DMA,
            pltpu.SemaphoreType.DMA,
            pltpu.SemaphoreType.DMA,
            pltpu.SemaphoreType.DMA,
            pltpu.SemaphoreType.DMA,
            pltpu.SemaphoreType.DMA,
        ],
    )
    def sc_scatter(tab, srcs, dsts, zeros, out,
                   srcv0, srcv1, dstv0, dstv1, rows0, rows1, acc,
                   gsem00, gsem01, gsem10, gsem11, isem0, isem1):
        c = lax.axis_index("c")
        s = lax.axis_index("s")
        w = c * 16 + s
        srcv = (srcv0, srcv1)
        dstv = (dstv0, dstv1)
        rows = (rows0, rows1)
        gsem = ((gsem00, gsem01), (gsem10, gsem11))
        isem = (isem0, isem1)
        HC = CHUNK // 2

        def fire_gather(idx_ref, t, b):
            pltpu.async_copy(tab.at[idx_ref.at[t, pl.ds(0, HC)]],
                             rows[b].at[pl.ds(0, HC)], gsem[b][0])
            pltpu.async_copy(tab.at[idx_ref.at[t, pl.ds(HC, HC)]],
                             rows[b].at[pl.ds(HC, HC)], gsem[b][1])

        def wait_gather(idx_ref, t, b):
            pltpu.make_async_copy(tab.at[idx_ref.at[t, pl.ds(0, HC)]],
                                  rows[b].at[pl.ds(0, HC)], gsem[b][0]).wait()
            pltpu.make_async_copy(tab.at[idx_ref.at[t, pl.ds(HC, HC)]],
                                  rows[b].at[pl.ds(HC, HC)], gsem[b][1]).wait()

        pltpu.sync_copy(zeros.at[pl.ds(s * _ZROWS, _ZROWS)],
                        acc.at[pl.ds(s * _ZROWS, _ZROWS)])
        pltpu.sync_copy(srcs.at[w, pl.ds(0, GB)], srcv0)
        pltpu.sync_copy(dsts.at[w, pl.ds(0, GB)], dstv0)
        pltpu.async_copy(srcs.at[w, pl.ds(GB, GB)], srcv1, isem1)
        pltpu.async_copy(dsts.at[w, pl.ds(GB, GB)], dstv1, isem1)
        plsc.subcore_barrier()

        fire_gather(srcv0, 0, 0)
        fire_gather(srcv0, 1, 1)

        def run_group(g, slot):
            osl = 1 - slot
            for t in range(GB):
                b = t % 2
                wait_gather(srcv[slot], t, b)
                pltpu.sync_copy(rows[b], acc.at[dstv[slot].at[t]], add=True)
                if t < GB - 2:
                    fire_gather(srcv[slot], t + 2, b)
                else:
                    @pl.when(g + 1 < NGRP)
                    def _():
                        if t == GB - 2:
                            pltpu.make_async_copy(srcs.at[w, pl.ds(0, GB)],
                                                  srcv[osl], isem[osl]).wait()
                            pltpu.make_async_copy(dsts.at[w, pl.ds(0, GB)],
                                                  dstv[osl], isem[osl]).wait()
                        fire_gather(srcv[osl], t + 2 - GB, b)
            @pl.when(g + 2 < NGRP)
            def _():
                off = (g + 2) * GB
                pltpu.async_copy(srcs.at[w, pl.ds(off, GB)], srcv[slot],
                                 isem[slot])
                pltpu.async_copy(dsts.at[w, pl.ds(off, GB)], dstv[slot],
                                 isem[slot])

        def body(p, carry):
            run_group(2 * p, 0)
            run_group(2 * p + 1, 1)
            return carry

        lax.fori_loop(0, NGRP // 2, body, 0)
        plsc.subcore_barrier()
        pltpu.sync_copy(acc.at[pl.ds(s * _ZROWS, _ZROWS)],
                        out.at[c, pl.ds(s * _ZROWS, _ZROWS)])

    @functools.partial(
        pl.kernel,
        out_type=jax.ShapeDtypeStruct((2, NPAD, DEGW), jnp.float32),
        mesh=mesh,
        scratch_types=[
            pltpu.VMEM((NCHUNK, CHUNK), jnp.int32),
            pltpu.VMEM((CHUNK, DEGW), jnp.float32),
            pltpu.VMEM_SHARED((NPAD, DEGW), jnp.float32),
            pltpu.SemaphoreType.DMA,
        ],
    )
    def sc_degree(srcs, zeros, ones, out, srcv, onesv, acc, sem):
        c = lax.axis_index("c")
        s = lax.axis_index("s")
        w = c * 16 + s
        win = 8
        pltpu.sync_copy(zeros.at[pl.ds(s * _ZROWS, _ZROWS)],
                        acc.at[pl.ds(s * _ZROWS, _ZROWS)])
        pltpu.sync_copy(srcs.at[w], srcv)
        pltpu.sync_copy(ones, onesv)
        plsc.subcore_barrier()

        def body(j, carry):
            pltpu.sync_copy(onesv, acc.at[srcv.at[j]], add=True)
            return carry

        lax.fori_loop(0, NCHUNK, body, 0)
        plsc.subcore_barrier()
        pltpu.sync_copy(acc.at[pl.ds(s * _ZROWS, _ZROWS)],
                        out.at[c, pl.ds(s * _ZROWS, _ZROWS)])

    return sc_scatter, sc_degree


def _dot(a, b):
    return jnp.dot(a, b, preferred_element_type=jnp.float32,
                   precision=lax.Precision.HIGHEST)


def _tca_body(v_ref, w_ref, deg_ref, out0_ref, u0_ref, dinv_ref):
    deg = deg_ref[0, :, 0:1] + deg_ref[1, :, 0:1]
    dinv = jnp.where(deg > 0, lax.rsqrt(jnp.maximum(deg, 1e-12)), 0.0)
    dinv_b = jnp.broadcast_to(dinv, (NB, D))
    x = v_ref[...]
    out0_ref[...] = _dot(x, w_ref[...])
    u0_ref[...] = dinv_b * x
    dinv_ref[...] = dinv_b


_tca1 = pl.pallas_call(
    _tca_body,
    grid=(GRID,),
    in_specs=[
        pl.BlockSpec((NB, D), lambda i: (i, 0)),
        pl.BlockSpec((D, D), lambda i: (0, 0)),
        pl.BlockSpec((2, NB, DEGW), lambda i: (0, i, 0)),
    ],
    out_specs=[
        pl.BlockSpec((NB, D), lambda i: (i, 0)),
        pl.BlockSpec((NB, D), lambda i: (i, 0)),
        pl.BlockSpec((NB, D), lambda i: (i, 0)),
    ],
    out_shape=[
        jax.ShapeDtypeStruct((N, D), jnp.float32),
        jax.ShapeDtypeStruct((NPAD, D), jnp.float32),
        jax.ShapeDtypeStruct((N, D), jnp.float32),
    ],
)


def _make_tcb(hh, do):
    di = hh * D

    def body(*refs):
        s_refs = refs[:hh]
        dinv_ref, out0_ref, w_ref = refs[hh:hh + 3]
        out1_ref = refs[hh + 3]
        u_refs = refs[hh + 4:]
        dinv = dinv_ref[...]
        txh = []
        for h in range(hh):
            sh = s_refs[h][...]
            t = -dinv * (sh[0] + sh[1])
            txh.append(t)
            u_refs[h][...] = dinv * t
        tx1 = jnp.concatenate(txh, axis=1) if hh > 1 else txh[0]
        out1_ref[...] = out0_ref[...] + _dot(tx1, w_ref[...])

    return pl.pallas_call(
        body,
        grid=(GRID,),
        in_specs=(
            [pl.BlockSpec((2, NB, D), lambda i: (0, i, 0)) for _ in range(hh)]
            + [
                pl.BlockSpec((NB, D), lambda i: (i, 0)),
                pl.BlockSpec((NB, do), lambda i: (i, 0)),
                pl.BlockSpec((di, do), lambda i: (0, 0)),
            ]
        ),
        out_specs=(
            [pl.BlockSpec((NB, do), lambda i: (i, 0))]
            + [pl.BlockSpec((NB, D), lambda i: (i, 0)) for _ in range(hh)]
        ),
        out_shape=(
            [jax.ShapeDtypeStruct((N, do), jnp.float32)]
            + [jax.ShapeDtypeStruct((NPAD, D), jnp.float32) for _ in range(hh)]
        ),
    )


def _make_tcc(hh, do, dn):
    di = hh * D
    ho = do // D

    def body(*refs):
        s_refs = refs[:hh]
        dinv_ref, out1_ref, x0_ref, w_ref, b_ref = refs[hh:hh + 5]
        k = hh + 5
        wn_ref = refs[k] if dn else None
        k += 1 if dn else 0
        h_ref = refs[k]
        u_refs = refs[k + 1:k + 1 + (ho if dn else 0)]
        o0_ref = refs[k + 1 + ho] if dn else None

        dinv = dinv_ref[...]
        x0 = x0_ref[...]
        txh = []
        for h in range(hh):
            sh = s_refs[h][...]
            txh.append(-2.0 * dinv * (sh[0] + sh[1]) - x0[:, h * D:(h + 1) * D])
        tx2 = jnp.concatenate(txh, axis=1) if hh > 1 else txh[0]
        hm = jax.nn.relu(out1_ref[...] + _dot(tx2, w_ref[...]) + b_ref[...])
        h_ref[...] = hm
        if dn:
            for h in range(ho):
                u_refs[h][...] = dinv * hm[:, h * D:(h + 1) * D]
            o0_ref[...] = _dot(hm, wn_ref[...])

    in_specs = (
        [pl.BlockSpec((2, NB, D), lambda i: (0, i, 0)) for _ in range(hh)]
        + [
            pl.BlockSpec((NB, D), lambda i: (i, 0)),
            pl.BlockSpec((NB, do), lambda i: (i, 0)),
            pl.BlockSpec((NB, di), lambda i: (i, 0)),
            pl.BlockSpec((di, do), lambda i: (0, 0)),
            pl.BlockSpec((1, do), lambda i: (0, 0)),
        ]
    )
    out_specs = [pl.BlockSpec((NB, do), lambda i: (i, 0))]
    out_shape = [jax.ShapeDtypeStruct((N, do), jnp.float32)]
    if dn:
        in_specs.append(pl.BlockSpec((do, dn), lambda i: (0, 0)))
        out_specs += [pl.BlockSpec((NB, D), lambda i: (i, 0)) for _ in range(ho)]
        out_shape += [jax.ShapeDtypeStruct((NPAD, D), jnp.float32) for _ in range(ho)]
        out_specs.append(pl.BlockSpec((NB, dn), lambda i: (i, 0)))
        out_shape.append(jax.ShapeDtypeStruct((N, dn), jnp.float32))

    return pl.pallas_call(body, grid=(GRID,), in_specs=in_specs,
                          out_specs=out_specs, out_shape=out_shape)


_tcb1 = _make_tcb(1, 128)
_tcc1 = _make_tcc(1, 128, 256)
_tcb2 = _make_tcb(1, 256)
_tcc2 = _make_tcc(1, 256, 512)
_tcb3 = _make_tcb(2, 512)
_tcc3 = _make_tcc(2, 512, 0)


def kernel(v, edge_index, W1, b1, W2, b2, W3, b3):
    e = edge_index.shape[1]
    pad = jnp.full((EPAD - e,), N, dtype=jnp.int32)
    srcs = jnp.concatenate([edge_index[0].astype(jnp.int32), pad]
                           ).reshape(NTILES, NCHUNK, CHUNK)
    dsts = jnp.concatenate([edge_index[1].astype(jnp.int32), pad]
                           ).reshape(NTILES, NCHUNK, CHUNK)
    zeros128 = jnp.zeros((NPAD, D), jnp.float32)
    zerosw = jnp.zeros((NPAD, DEGW), jnp.float32)
    onesw = jnp.ones((CHUNK, DEGW), jnp.float32)

    _sc_scatter, _sc_degree = _sc_kernels()
    deg = _sc_degree(srcs, zerosw, onesw)
    out0, u0, dinv = _tca1(v, W1[0], deg)

    s0 = _sc_scatter(u0, srcs, dsts, zeros128)
    out1, u1 = _tcb1(s0, dinv, out0, W1[1])
    s1 = _sc_scatter(u1, srcs, dsts, zeros128)
    h1, u0b, out0b = _tcc1(s1, dinv, out1, v, W1[2], b1.reshape(1, -1), W2[0])

    s0 = _sc_scatter(u0b, srcs, dsts, zeros128)
    out1, u1 = _tcb2(s0, dinv, out0b, W2[1])
    s1 = _sc_scatter(u1, srcs, dsts, zeros128)
    h2, u0c0, u0c1, out0c = _tcc2(s1, dinv, out1, h1, W2[2], b2.reshape(1, -1),
                                  W3[0])

    s0a = _sc_scatter(u0c0, srcs, dsts, zeros128)
    s0b = _sc_scatter(u0c1, srcs, dsts, zeros128)
    out1, u1a, u1b = _tcb3(s0a, s0b, dinv, out0c, W3[1])
    s1a = _sc_scatter(u1a, srcs, dsts, zeros128)
    s1b = _sc_scatter(u1b, srcs, dsts, zeros128)
    h3 = _tcc3(s1a, s1b, dinv, out1, h2, W3[2], b3.reshape(1, -1))
    return h3[0]


# speedup vs baseline: 1.0008x; 1.0008x over previous
import functools

import jax
import jax.numpy as jnp
from jax import lax
from jax.experimental import pallas as pl
from jax.experimental.pallas import tpu as pltpu
from jax.experimental.pallas import tpu_sc as plsc

N = 10000
NPAD = 10112
D = 128
DEGW = 128
CHUNK = 128
NCHUNK = 80
NTILES = 32
EPAD = NTILES * NCHUNK * CHUNK
NB = 1000
GRID = N // NB

_ZROWS = NPAD // 16


@functools.cache
def _sc_kernels():
    mesh = plsc.VectorSubcoreMesh(core_axis_name="c", subcore_axis_name="s")

    GB = 8
    NGRP = NCHUNK // GB

    @functools.partial(
        pl.kernel,
        out_type=jax.ShapeDtypeStruct((2, NPAD, D), jnp.float32),
        mesh=mesh,
        scratch_types=[
            pltpu.VMEM((GB, CHUNK), jnp.int32),
            pltpu.VMEM((GB, CHUNK), jnp.int32),
            pltpu.VMEM((GB, CHUNK), jnp.int32),
            pltpu.VMEM((GB, CHUNK), jnp.int32),
            pltpu.VMEM((CHUNK, D), jnp.float32),
            pltpu.VMEM((CHUNK, D), jnp.float32),
            pltpu.VMEM_SHARED((NPAD, D), jnp.float32),
            pltpu.SemaphoreType.DMA,
            pltpu.SemaphoreType.DMA,
            pltpu.SemaphoreType.DMA,
            pltpu.SemaphoreType.DMA,
            pltpu.SemaphoreType.DMA,
            pltpu.SemaphoreType.DMA,
        ],
    )
    def sc_scatter(tab, srcs, dsts, zeros, out,
                   srcv0, srcv1, dstv0, dstv1, rows0, rows1, acc,
                   gsem00, gsem01, gsem10, gsem11, isem0, isem1):
        c = lax.axis_index("c")
        s = lax.axis_index("s")
        w = c * 16 + s
        srcv = (srcv0, srcv1)
        dstv = (dstv0, dstv1)
        rows = (rows0, rows1)
        gsem = ((gsem00, gsem01), (gsem10, gsem11))
        isem = (isem0, isem1)
        HC = CHUNK // 2

        def fire_gather(idx_ref, t, b):
            pltpu.async_copy(tab.at[idx_ref.at[t, pl.ds(0, HC)]],
                             rows[b].at[pl.ds(0, HC)], gsem[b][0])
            pltpu.async_copy(tab.at[idx_ref.at[t, pl.ds(HC, HC)]],
                             rows[b].at[pl.ds(HC, HC)], gsem[b][1])

        def wait_gather(idx_ref, t, b):
            pltpu.make_async_copy(tab.at[idx_ref.at[t, pl.ds(0, HC)]],
                                  rows[b].at[pl.ds(0, HC)], gsem[b][0]).wait()
            pltpu.make_async_copy(tab.at[idx_ref.at[t, pl.ds(HC, HC)]],
                                  rows[b].at[pl.ds(HC, HC)], gsem[b][1]).wait()

        pltpu.sync_copy(zeros.at[pl.ds(s * _ZROWS, _ZROWS)],
                        acc.at[pl.ds(s * _ZROWS, _ZROWS)])
        pltpu.sync_copy(srcs.at[w, pl.ds(0, GB)], srcv0)
        pltpu.sync_copy(dsts.at[w, pl.ds(0, GB)], dstv0)
        pltpu.async_copy(srcs.at[w, pl.ds(GB, GB)], srcv1, isem1)
        pltpu.async_copy(dsts.at[w, pl.ds(GB, GB)], dstv1, isem1)
        plsc.subcore_barrier()

        fire_gather(srcv0, 0, 0)
        fire_gather(srcv0, 1, 1)

        def run_group(g, slot):
            osl = 1 - slot
            for t in range(GB):
                b = t % 2
                wait_gather(srcv[slot], t, b)
                pltpu.sync_copy(rows[b], acc.at[dstv[slot].at[t]], add=True)
                if t < GB - 2:
                    fire_gather(srcv[slot], t + 2, b)
                else:
                    @pl.when(g + 1 < NGRP)
                    def _():
                        if t == GB - 2:
                            pltpu.make_async_copy(srcs.at[w, pl.ds(0, GB)],
                                                  srcv[osl], isem[osl]).wait()
                            pltpu.make_async_copy(dsts.at[w, pl.ds(0, GB)],
                                                  dstv[osl], isem[osl]).wait()
                        fire_gather(srcv[osl], t + 2 - GB, b)
            @pl.when(g + 2 < NGRP)
            def _():
                off = (g + 2) * GB
                pltpu.async_copy(srcs.at[w, pl.ds(off, GB)], srcv[slot],
                                 isem[slot])
                pltpu.async_copy(dsts.at[w, pl.ds(off, GB)], dstv[slot],
                                 isem[slot])

        def body(p, carry):
            run_group(2 * p, 0)
            run_group(2 * p + 1, 1)
            return carry

        lax.fori_loop(0, NGRP // 2, body, 0)
        plsc.subcore_barrier()
        pltpu.sync_copy(acc.at[pl.ds(s * _ZROWS, _ZROWS)],
                        out.at[c, pl.ds(s * _ZROWS, _ZROWS)])

    @functools.partial(
        pl.kernel,
        out_type=jax.ShapeDtypeStruct((2, NPAD, DEGW), jnp.float32),
        mesh=mesh,
        scratch_types=[
            pltpu.VMEM((NCHUNK, CHUNK), jnp.int32),
            pltpu.VMEM((CHUNK, DEGW), jnp.float32),
            pltpu.VMEM_SHARED((NPAD, DEGW), jnp.float32),
        ],
    )
    def sc_degree(srcs, zeros, ones, out, srcv, onesv, acc):
        c = lax.axis_index("c")
        s = lax.axis_index("s")
        w = c * 16 + s
        pltpu.sync_copy(zeros.at[pl.ds(s * _ZROWS, _ZROWS)],
                        acc.at[pl.ds(s * _ZROWS, _ZROWS)])
        pltpu.sync_copy(srcs.at[w], srcv)
        pltpu.sync_copy(ones, onesv)
        plsc.subcore_barrier()

        def body(j, carry):
            pltpu.sync_copy(onesv, acc.at[srcv.at[j]], add=True)
            return carry

        lax.fori_loop(0, NCHUNK, body, 0)
        plsc.subcore_barrier()
        pltpu.sync_copy(acc.at[pl.ds(s * _ZROWS, _ZROWS)],
                        out.at[c, pl.ds(s * _ZROWS, _ZROWS)])

    return sc_scatter, sc_degree


def _dot(a, b):
    return jnp.dot(a, b, preferred_element_type=jnp.float32,
                   precision=lax.Precision.HIGHEST)


def _tca_body(v_ref, w_ref, deg_ref, out0_ref, u0_ref, dinv_ref):
    deg = deg_ref[0, :, 0:1] + deg_ref[1, :, 0:1]
    dinv = jnp.where(deg > 0, lax.rsqrt(jnp.maximum(deg, 1e-12)), 0.0)
    dinv_b = jnp.broadcast_to(dinv, (NB, D))
    x = v_ref[...]
    out0_ref[...] = _dot(x, w_ref[...])
    u0_ref[...] = dinv_b * x
    dinv_ref[...] = dinv_b


_tca1 = pl.pallas_call(
    _tca_body,
    grid=(GRID,),
    in_specs=[
        pl.BlockSpec((NB, D), lambda i: (i, 0)),
        pl.BlockSpec((D, D), lambda i: (0, 0)),
        pl.BlockSpec((2, NB, DEGW), lambda i: (0, i, 0)),
    ],
    out_specs=[
        pl.BlockSpec((NB, D), lambda i: (i, 0)),
        pl.BlockSpec((NB, D), lambda i: (i, 0)),
        pl.BlockSpec((NB, D), lambda i: (i, 0)),
    ],
    out_shape=[
        jax.ShapeDtypeStruct((N, D), jnp.float32),
        jax.ShapeDtypeStruct((NPAD, D), jnp.float32),
        jax.ShapeDtypeStruct((N, D), jnp.float32),
    ],
)


def _make_tcb(hh, do):
    di = hh * D

    def body(*refs):
        s_refs = refs[:hh]
        dinv_ref, out0_ref, w_ref = refs[hh:hh + 3]
        out1_ref = refs[hh + 3]
        u_refs = refs[hh + 4:]
        dinv = dinv_ref[...]
        txh = []
        for h in range(hh):
            sh = s_refs[h][...]
            t = -dinv * (sh[0] + sh[1])
            txh.append(t)
            u_refs[h][...] = dinv * t
        tx1 = jnp.concatenate(txh, axis=1) if hh > 1 else txh[0]
        out1_ref[...] = out0_ref[...] + _dot(tx1, w_ref[...])

    return pl.pallas_call(
        body,
        grid=(GRID,),
        in_specs=(
            [pl.BlockSpec((2, NB, D), lambda i: (0, i, 0)) for _ in range(hh)]
            + [
                pl.BlockSpec((NB, D), lambda i: (i, 0)),
                pl.BlockSpec((NB, do), lambda i: (i, 0)),
                pl.BlockSpec((di, do), lambda i: (0, 0)),
            ]
        ),
        out_specs=(
            [pl.BlockSpec((NB, do), lambda i: (i, 0))]
            + [pl.BlockSpec((NB, D), lambda i: (i, 0)) for _ in range(hh)]
        ),
        out_shape=(
            [jax.ShapeDtypeStruct((N, do), jnp.float32)]
            + [jax.ShapeDtypeStruct((NPAD, D), jnp.float32) for _ in range(hh)]
        ),
    )


def _make_tcc(hh, do, dn):
    di = hh * D
    ho = do // D

    def body(*refs):
        s_refs = refs[:hh]
        dinv_ref, out1_ref, x0_ref, w_ref, b_ref = refs[hh:hh + 5]
        k = hh + 5
        wn_ref = refs[k] if dn else None
        k += 1 if dn else 0
        h_ref = refs[k]
        u_refs = refs[k + 1:k + 1 + (ho if dn else 0)]
        o0_ref = refs[k + 1 + ho] if dn else None

        dinv = dinv_ref[...]
        x0 = x0_ref[...]
        txh = []
        for h in range(hh):
            sh = s_refs[h][...]
            txh.append(-2.0 * dinv * (sh[0] + sh[1]) - x0[:, h * D:(h + 1) * D])
        tx2 = jnp.concatenate(txh, axis=1) if hh > 1 else txh[0]
        hm = jax.nn.relu(out1_ref[...] + _dot(tx2, w_ref[...]) + b_ref[...])
        h_ref[...] = hm
        if dn:
            for h in range(ho):
                u_refs[h][...] = dinv * hm[:, h * D:(h + 1) * D]
            o0_ref[...] = _dot(hm, wn_ref[...])

    in_specs = (
        [pl.BlockSpec((2, NB, D), lambda i: (0, i, 0)) for _ in range(hh)]
        + [
            pl.BlockSpec((NB, D), lambda i: (i, 0)),
            pl.BlockSpec((NB, do), lambda i: (i, 0)),
            pl.BlockSpec((NB, di), lambda i: (i, 0)),
            pl.BlockSpec((di, do), lambda i: (0, 0)),
            pl.BlockSpec((1, do), lambda i: (0, 0)),
        ]
    )
    out_specs = [pl.BlockSpec((NB, do), lambda i: (i, 0))]
    out_shape = [jax.ShapeDtypeStruct((N, do), jnp.float32)]
    if dn:
        in_specs.append(pl.BlockSpec((do, dn), lambda i: (0, 0)))
        out_specs += [pl.BlockSpec((NB, D), lambda i: (i, 0)) for _ in range(ho)]
        out_shape += [jax.ShapeDtypeStruct((NPAD, D), jnp.float32) for _ in range(ho)]
        out_specs.append(pl.BlockSpec((NB, dn), lambda i: (i, 0)))
        out_shape.append(jax.ShapeDtypeStruct((N, dn), jnp.float32))

    return pl.pallas_call(body, grid=(GRID,), in_specs=in_specs,
                          out_specs=out_specs, out_shape=out_shape)


_tcb1 = _make_tcb(1, 128)
_tcc1 = _make_tcc(1, 128, 256)
_tcb2 = _make_tcb(1, 256)
_tcc2 = _make_tcc(1, 256, 512)
_tcb3 = _make_tcb(2, 512)
_tcc3 = _make_tcc(2, 512, 0)


def kernel(v, edge_index, W1, b1, W2, b2, W3, b3):
    e = edge_index.shape[1]
    pad = jnp.full((EPAD - e,), N, dtype=jnp.int32)
    srcs = jnp.concatenate([edge_index[0].astype(jnp.int32), pad]
                           ).reshape(NTILES, NCHUNK, CHUNK)
    dsts = jnp.concatenate([edge_index[1].astype(jnp.int32), pad]
                           ).reshape(NTILES, NCHUNK, CHUNK)
    zeros128 = jnp.zeros((NPAD, D), jnp.float32)
    zerosw = jnp.zeros((NPAD, DEGW), jnp.float32)
    onesw = jnp.ones((CHUNK, DEGW), jnp.float32)

    _sc_scatter, _sc_degree = _sc_kernels()
    deg = _sc_degree(srcs, zerosw, onesw)
    out0, u0, dinv = _tca1(v, W1[0], deg)

    s0 = _sc_scatter(u0, srcs, dsts, zeros128)
    out1, u1 = _tcb1(s0, dinv, out0, W1[1])
    s1 = _sc_scatter(u1, srcs, dsts, zeros128)
    h1, u0b, out0b = _tcc1(s1, dinv, out1, v, W1[2], b1.reshape(1, -1), W2[0])

    s0 = _sc_scatter(u0b, srcs, dsts, zeros128)
    out1, u1 = _tcb2(s0, dinv, out0b, W2[1])
    s1 = _sc_scatter(u1, srcs, dsts, zeros128)
    h2, u0c0, u0c1, out0c = _tcc2(s1, dinv, out1, h1, W2[2], b2.reshape(1, -1),
                                  W3[0])

    s0a = _sc_scatter(u0c0, srcs, dsts, zeros128)
    s0b = _sc_scatter(u0c1, srcs, dsts, zeros128)
    out1, u1a, u1b = _tcb3(s0a, s0b, dinv, out0c, W3[1])
    s1a = _sc_scatter(u1a, srcs, dsts, zeros128)
    s1b = _sc_scatter(u1b, srcs, dsts, zeros128)
    h3 = _tcc3(s1a, s1b, dinv, out1, h2, W3[2], b3.reshape(1, -1))
    return h3[0]


# speedup vs baseline: 1.0008x; 1.0000x over previous
import functools

import jax
import jax.numpy as jnp
from jax import lax
from jax.experimental import pallas as pl
from jax.experimental.pallas import tpu as pltpu
from jax.experimental.pallas import tpu_sc as plsc

N = 10000
NPAD = 10112
D = 128
DEGW = 128
CHUNK = 128
NCHUNK = 80
NTILES = 32
EPAD = NTILES * NCHUNK * CHUNK
NB = 1000
GRID = N // NB

_ZROWS = NPAD // 16


@functools.cache
def _sc_kernels():
    mesh = plsc.VectorSubcoreMesh(core_axis_name="c", subcore_axis_name="s")

    GB = 8
    NGRP = NCHUNK // GB

    @functools.partial(
        pl.kernel,
        out_type=jax.ShapeDtypeStruct((2, NPAD, D), jnp.float32),
        mesh=mesh,
        scratch_types=[
            pltpu.VMEM((GB, CHUNK), jnp.int32),
            pltpu.VMEM((GB, CHUNK), jnp.int32),
            pltpu.VMEM((GB, CHUNK), jnp.int32),
            pltpu.VMEM((GB, CHUNK), jnp.int32),
            pltpu.VMEM((CHUNK, D), jnp.float32),
            pltpu.VMEM((CHUNK, D), jnp.float32),
            pltpu.VMEM_SHARED((NPAD, D), jnp.float32),
            pltpu.SemaphoreType.DMA,
            pltpu.SemaphoreType.DMA,
            pltpu.SemaphoreType.DMA,
            pltpu.SemaphoreType.DMA,
            pltpu.SemaphoreType.DMA,
            pltpu.SemaphoreType.DMA,
        ],
    )
    def sc_scatter(tab, srcs, dsts, zeros, out,
                   srcv0, srcv1, dstv0, dstv1, rows0, rows1, acc,
                   gsem00, gsem01, gsem10, gsem11, isem0, isem1):
        c = lax.axis_index("c")
        s = lax.axis_index("s")
        w = c * 16 + s
        srcv = (srcv0, srcv1)
        dstv = (dstv0, dstv1)
        rows = (rows0, rows1)
        gsem = ((gsem00, gsem01), (gsem10, gsem11))
        isem = (isem0, isem1)
        HC = CHUNK // 2

        def fire_gather(idx_ref, t, b):
            pltpu.async_copy(tab.at[idx_ref.at[t, pl.ds(0, HC)]],
                             rows[b].at[pl.ds(0, HC)], gsem[b][0])
            pltpu.async_copy(tab.at[idx_ref.at[t, pl.ds(HC, HC)]],
                             rows[b].at[pl.ds(HC, HC)], gsem[b][1], priority=1)

        def wait_gather(idx_ref, t, b):
            pltpu.make_async_copy(tab.at[idx_ref.at[t, pl.ds(0, HC)]],
                                  rows[b].at[pl.ds(0, HC)], gsem[b][0]).wait()
            pltpu.make_async_copy(tab.at[idx_ref.at[t, pl.ds(HC, HC)]],
                                  rows[b].at[pl.ds(HC, HC)], gsem[b][1]).wait()

        pltpu.sync_copy(zeros.at[pl.ds(s * _ZROWS, _ZROWS)],
                        acc.at[pl.ds(s * _ZROWS, _ZROWS)])
        pltpu.sync_copy(srcs.at[w, pl.ds(0, GB)], srcv0)
        pltpu.sync_copy(dsts.at[w, pl.ds(0, GB)], dstv0)
        pltpu.async_copy(srcs.at[w, pl.ds(GB, GB)], srcv1, isem1)
        pltpu.async_copy(dsts.at[w, pl.ds(GB, GB)], dstv1, isem1)
        plsc.subcore_barrier()

        fire_gather(srcv0, 0, 0)
        fire_gather(srcv0, 1, 1)

        def run_group(g, slot):
            osl = 1 - slot
            for t in range(GB):
                b = t % 2
                wait_gather(srcv[slot], t, b)
                pltpu.sync_copy(rows[b], acc.at[dstv[slot].at[t]], add=True)
                if t < GB - 2:
                    fire_gather(srcv[slot], t + 2, b)
                else:
                    @pl.when(g + 1 < NGRP)
                    def _():
                        if t == GB - 2:
                            pltpu.make_async_copy(srcs.at[w, pl.ds(0, GB)],
                                                  srcv[osl], isem[osl]).wait()
                            pltpu.make_async_copy(dsts.at[w, pl.ds(0, GB)],
                                                  dstv[osl], isem[osl]).wait()
                        fire_gather(srcv[osl], t + 2 - GB, b)
            @pl.when(g + 2 < NGRP)
            def _():
                off = (g + 2) * GB
                pltpu.async_copy(srcs.at[w, pl.ds(off, GB)], srcv[slot],
                                 isem[slot])
                pltpu.async_copy(dsts.at[w, pl.ds(off, GB)], dstv[slot],
                                 isem[slot])

        def body(p, carry):
            run_group(2 * p, 0)
            run_group(2 * p + 1, 1)
            return carry

        lax.fori_loop(0, NGRP // 2, body, 0)
        plsc.subcore_barrier()
        pltpu.sync_copy(acc.at[pl.ds(s * _ZROWS, _ZROWS)],
                        out.at[c, pl.ds(s * _ZROWS, _ZROWS)])

    @functools.partial(
        pl.kernel,
        out_type=jax.ShapeDtypeStruct((2, NPAD, DEGW), jnp.float32),
        mesh=mesh,
        scratch_types=[
            pltpu.VMEM((NCHUNK, CHUNK), jnp.int32),
            pltpu.VMEM((CHUNK, DEGW), jnp.float32),
            pltpu.VMEM_SHARED((NPAD, DEGW), jnp.float32),
        ],
    )
    def sc_degree(srcs, zeros, ones, out, srcv, onesv, acc):
        c = lax.axis_index("c")
        s = lax.axis_index("s")
        w = c * 16 + s
        pltpu.sync_copy(zeros.at[pl.ds(s * _ZROWS, _ZROWS)],
                        acc.at[pl.ds(s * _ZROWS, _ZROWS)])
        pltpu.sync_copy(srcs.at[w], srcv)
        pltpu.sync_copy(ones, onesv)
        plsc.subcore_barrier()

        def body(j, carry):
            pltpu.sync_copy(onesv, acc.at[srcv.at[j]], add=True)
            return carry

        lax.fori_loop(0, NCHUNK, body, 0)
        plsc.subcore_barrier()
        pltpu.sync_copy(acc.at[pl.ds(s * _ZROWS, _ZROWS)],
                        out.at[c, pl.ds(s * _ZROWS, _ZROWS)])

    return sc_scatter, sc_degree


def _dot(a, b):
    return jnp.dot(a, b, preferred_element_type=jnp.float32,
                   precision=lax.Precision.HIGHEST)


def _tca_body(v_ref, w_ref, deg_ref, out0_ref, u0_ref, dinv_ref):
    deg = deg_ref[0, :, 0:1] + deg_ref[1, :, 0:1]
    dinv = jnp.where(deg > 0, lax.rsqrt(jnp.maximum(deg, 1e-12)), 0.0)
    dinv_b = jnp.broadcast_to(dinv, (NB, D))
    x = v_ref[...]
    out0_ref[...] = _dot(x, w_ref[...])
    u0_ref[...] = dinv_b * x
    dinv_ref[...] = dinv_b


_tca1 = pl.pallas_call(
    _tca_body,
    grid=(GRID,),
    in_specs=[
        pl.BlockSpec((NB, D), lambda i: (i, 0)),
        pl.BlockSpec((D, D), lambda i: (0, 0)),
        pl.BlockSpec((2, NB, DEGW), lambda i: (0, i, 0)),
    ],
    out_specs=[
        pl.BlockSpec((NB, D), lambda i: (i, 0)),
        pl.BlockSpec((NB, D), lambda i: (i, 0)),
        pl.BlockSpec((NB, D), lambda i: (i, 0)),
    ],
    out_shape=[
        jax.ShapeDtypeStruct((N, D), jnp.float32),
        jax.ShapeDtypeStruct((NPAD, D), jnp.float32),
        jax.ShapeDtypeStruct((N, D), jnp.float32),
    ],
)


def _make_tcb(hh, do):
    di = hh * D

    def body(*refs):
        s_refs = refs[:hh]
        dinv_ref, out0_ref, w_ref = refs[hh:hh + 3]
        out1_ref = refs[hh + 3]
        u_refs = refs[hh + 4:]
        dinv = dinv_ref[...]
        txh = []
        for h in range(hh):
            sh = s_refs[h][...]
            t = -dinv * (sh[0] + sh[1])
            txh.append(t)
            u_refs[h][...] = dinv * t
        tx1 = jnp.concatenate(txh, axis=1) if hh > 1 else txh[0]
        out1_ref[...] = out0_ref[...] + _dot(tx1, w_ref[...])

    return pl.pallas_call(
        body,
        grid=(GRID,),
        in_specs=(
            [pl.BlockSpec((2, NB, D), lambda i: (0, i, 0)) for _ in range(hh)]
            + [
                pl.BlockSpec((NB, D), lambda i: (i, 0)),
                pl.BlockSpec((NB, do), lambda i: (i, 0)),
                pl.BlockSpec((di, do), lambda i: (0, 0)),
            ]
        ),
        out_specs=(
            [pl.BlockSpec((NB, do), lambda i: (i, 0))]
            + [pl.BlockSpec((NB, D), lambda i: (i, 0)) for _ in range(hh)]
        ),
        out_shape=(
            [jax.ShapeDtypeStruct((N, do), jnp.float32)]
            + [jax.ShapeDtypeStruct((NPAD, D), jnp.float32) for _ in range(hh)]
        ),
    )


def _make_tcc(hh, do, dn):
    di = hh * D
    ho = do // D

    def body(*refs):
        s_refs = refs[:hh]
        dinv_ref, out1_ref, x0_ref, w_ref, b_ref = refs[hh:hh + 5]
        k = hh + 5
        wn_ref = refs[k] if dn else None
        k += 1 if dn else 0
        h_ref = refs[k]
        u_refs = refs[k + 1:k + 1 + (ho if dn else 0)]
        o0_ref = refs[k + 1 + ho] if dn else None

        dinv = dinv_ref[...]
        x0 = x0_ref[...]
        txh = []
        for h in range(hh):
            sh = s_refs[h][...]
            txh.append(-2.0 * dinv * (sh[0] + sh[1]) - x0[:, h * D:(h + 1) * D])
        tx2 = jnp.concatenate(txh, axis=1) if hh > 1 else txh[0]
        hm = jax.nn.relu(out1_ref[...] + _dot(tx2, w_ref[...]) + b_ref[...])
        h_ref[...] = hm
        if dn:
            for h in range(ho):
                u_refs[h][...] = dinv * hm[:, h * D:(h + 1) * D]
            o0_ref[...] = _dot(hm, wn_ref[...])

    in_specs = (
        [pl.BlockSpec((2, NB, D), lambda i: (0, i, 0)) for _ in range(hh)]
        + [
            pl.BlockSpec((NB, D), lambda i: (i, 0)),
            pl.BlockSpec((NB, do), lambda i: (i, 0)),
            pl.BlockSpec((NB, di), lambda i: (i, 0)),
            pl.BlockSpec((di, do), lambda i: (0, 0)),
            pl.BlockSpec((1, do), lambda i: (0, 0)),
        ]
    )
    out_specs = [pl.BlockSpec((NB, do), lambda i: (i, 0))]
    out_shape = [jax.ShapeDtypeStruct((N, do), jnp.float32)]
    if dn:
        in_specs.append(pl.BlockSpec((do, dn), lambda i: (0, 0)))
        out_specs += [pl.BlockSpec((NB, D), lambda i: (i, 0)) for _ in range(ho)]
        out_shape += [jax.ShapeDtypeStruct((NPAD, D), jnp.float32) for _ in range(ho)]
        out_specs.append(pl.BlockSpec((NB, dn), lambda i: (i, 0)))
        out_shape.append(jax.ShapeDtypeStruct((N, dn), jnp.float32))

    return pl.pallas_call(body, grid=(GRID,), in_specs=in_specs,
                          out_specs=out_specs, out_shape=out_shape)


_tcb1 = _make_tcb(1, 128)
_tcc1 = _make_tcc(1, 128, 256)
_tcb2 = _make_tcb(1, 256)
_tcc2 = _make_tcc(1, 256, 512)
_tcb3 = _make_tcb(2, 512)
_tcc3 = _make_tcc(2, 512, 0)


def kernel(v, edge_index, W1, b1, W2, b2, W3, b3):
    e = edge_index.shape[1]
    pad = jnp.full((EPAD - e,), N, dtype=jnp.int32)
    srcs = jnp.concatenate([edge_index[0].astype(jnp.int32), pad]
                           ).reshape(NTILES, NCHUNK, CHUNK)
    dsts = jnp.concatenate([edge_index[1].astype(jnp.int32), pad]
                           ).reshape(NTILES, NCHUNK, CHUNK)
    zeros128 = jnp.zeros((NPAD, D), jnp.float32)
    zerosw = jnp.zeros((NPAD, DEGW), jnp.float32)
    onesw = jnp.ones((CHUNK, DEGW), jnp.float32)

    _sc_scatter, _sc_degree = _sc_kernels()
    deg = _sc_degree(srcs, zerosw, onesw)
    out0, u0, dinv = _tca1(v, W1[0], deg)

    s0 = _sc_scatter(u0, srcs, dsts, zeros128)
    out1, u1 = _tcb1(s0, dinv, out0, W1[1])
    s1 = _sc_scatter(u1, srcs, dsts, zeros128)
    h1, u0b, out0b = _tcc1(s1, dinv, out1, v, W1[2], b1.reshape(1, -1), W2[0])

    s0 = _sc_scatter(u0b, srcs, dsts, zeros128)
    out1, u1 = _tcb2(s0, dinv, out0b, W2[1])
    s1 = _sc_scatter(u1, srcs, dsts, zeros128)
    h2, u0c0, u0c1, out0c = _tcc2(s1, dinv, out1, h1, W2[2], b2.reshape(1, -1),
                                  W3[0])

    s0a = _sc_scatter(u0c0, srcs, dsts, zeros128)
    s0b = _sc_scatter(u0c1, srcs, dsts, zeros128)
    out1, u1a, u1b = _tcb3(s0a, s0b, dinv, out0c, W3[1])
    s1a = _sc_scatter(u1a, srcs, dsts, zeros128)
    s1b = _sc_scatter(u1b, srcs, dsts, zeros128)
    h3 = _tcc3(s1a, s1b, dinv, out1, h2, W3[2], b3.reshape(1, -1))
    return h3[0]


# speedup vs baseline: 1.0049x; 1.0041x over previous
import functools

import jax
import jax.numpy as jnp
from jax import lax
from jax.experimental import pallas as pl
from jax.experimental.pallas import tpu as pltpu
from jax.experimental.pallas import tpu_sc as plsc

N = 10000
NPAD = 10112
D = 128
DEGW = 128
CHUNK = 128
NCHUNK = 80
NTILES = 32
EPAD = NTILES * NCHUNK * CHUNK
NB = 1000
GRID = N // NB

_ZROWS = NPAD // 16


@functools.cache
def _sc_kernels():
    mesh = plsc.VectorSubcoreMesh(core_axis_name="c", subcore_axis_name="s")

    GB = 8
    NGRP = NCHUNK // GB

    @functools.partial(
        pl.kernel,
        out_type=jax.ShapeDtypeStruct((2, NPAD, D), jnp.float32),
        mesh=mesh,
        scratch_types=[
            pltpu.VMEM((GB, CHUNK), jnp.int32),
            pltpu.VMEM((GB, CHUNK), jnp.int32),
            pltpu.VMEM((GB, CHUNK), jnp.int32),
            pltpu.VMEM((GB, CHUNK), jnp.int32),
            pltpu.VMEM((CHUNK, D), jnp.float32),
            pltpu.VMEM((CHUNK, D), jnp.float32),
            pltpu.VMEM_SHARED((NPAD, D), jnp.float32),
            pltpu.SemaphoreType.DMA,
            pltpu.SemaphoreType.DMA,
            pltpu.SemaphoreType.DMA,
            pltpu.SemaphoreType.DMA,
            pltpu.SemaphoreType.DMA,
            pltpu.SemaphoreType.DMA,
        ],
    )
    def sc_scatter(tab, srcs, dsts, zeros, out,
                   srcv0, srcv1, dstv0, dstv1, rows0, rows1, acc,
                   gsem00, gsem01, gsem10, gsem11, isem0, isem1):
        c = lax.axis_index("c")
        s = lax.axis_index("s")
        w = c * 16 + s
        srcv = (srcv0, srcv1)
        dstv = (dstv0, dstv1)
        rows = (rows0, rows1)
        gsem = ((gsem00, gsem01), (gsem10, gsem11))
        isem = (isem0, isem1)
        HC = CHUNK // 2

        def fire_gather(idx_ref, t, b):
            pltpu.async_copy(tab.at[idx_ref.at[t, pl.ds(0, HC)]],
                             rows[b].at[pl.ds(0, HC)], gsem[b][0])
            pltpu.async_copy(tab.at[idx_ref.at[t, pl.ds(HC, HC)]],
                             rows[b].at[pl.ds(HC, HC)], gsem[b][1])

        def wait_gather(idx_ref, t, b):
            pltpu.make_async_copy(tab.at[idx_ref.at[t, pl.ds(0, HC)]],
                                  rows[b].at[pl.ds(0, HC)], gsem[b][0]).wait()
            pltpu.make_async_copy(tab.at[idx_ref.at[t, pl.ds(HC, HC)]],
                                  rows[b].at[pl.ds(HC, HC)], gsem[b][1]).wait()

        pltpu.sync_copy(srcs.at[w, pl.ds(0, GB)], srcv0)
        pltpu.sync_copy(dsts.at[w, pl.ds(0, GB)], dstv0)
        fire_gather(srcv0, 0, 0)
        fire_gather(srcv0, 1, 1)
        pltpu.async_copy(srcs.at[w, pl.ds(GB, GB)], srcv1, isem1)
        pltpu.async_copy(dsts.at[w, pl.ds(GB, GB)], dstv1, isem1)
        pltpu.sync_copy(zeros.at[pl.ds(s * _ZROWS, _ZROWS)],
                        acc.at[pl.ds(s * _ZROWS, _ZROWS)])
        plsc.subcore_barrier()

        def run_group(g, slot):
            osl = 1 - slot
            for t in range(GB):
                b = t % 2
                wait_gather(srcv[slot], t, b)
                pltpu.sync_copy(rows[b], acc.at[dstv[slot].at[t]], add=True)
                if t < GB - 2:
                    fire_gather(srcv[slot], t + 2, b)
                else:
                    @pl.when(g + 1 < NGRP)
                    def _():
                        if t == GB - 2:
                            pltpu.make_async_copy(srcs.at[w, pl.ds(0, GB)],
                                                  srcv[osl], isem[osl]).wait()
                            pltpu.make_async_copy(dsts.at[w, pl.ds(0, GB)],
                                                  dstv[osl], isem[osl]).wait()
                        fire_gather(srcv[osl], t + 2 - GB, b)
            @pl.when(g + 2 < NGRP)
            def _():
                off = (g + 2) * GB
                pltpu.async_copy(srcs.at[w, pl.ds(off, GB)], srcv[slot],
                                 isem[slot])
                pltpu.async_copy(dsts.at[w, pl.ds(off, GB)], dstv[slot],
                                 isem[slot])

        def body(p, carry):
            run_group(2 * p, 0)
            run_group(2 * p + 1, 1)
            return carry

        lax.fori_loop(0, NGRP // 2, body, 0)
        plsc.subcore_barrier()
        pltpu.sync_copy(acc.at[pl.ds(s * _ZROWS, _ZROWS)],
                        out.at[c, pl.ds(s * _ZROWS, _ZROWS)])

    @functools.partial(
        pl.kernel,
        out_type=jax.ShapeDtypeStruct((2, NPAD, DEGW), jnp.float32),
        mesh=mesh,
        scratch_types=[
            pltpu.VMEM((NCHUNK, CHUNK), jnp.int32),
            pltpu.VMEM((CHUNK, DEGW), jnp.float32),
            pltpu.VMEM_SHARED((NPAD, DEGW), jnp.float32),
        ],
    )
    def sc_degree(srcs, zeros, ones, out, srcv, onesv, acc):
        c = lax.axis_index("c")
        s = lax.axis_index("s")
        w = c * 16 + s
        pltpu.sync_copy(zeros.at[pl.ds(s * _ZROWS, _ZROWS)],
                        acc.at[pl.ds(s * _ZROWS, _ZROWS)])
        pltpu.sync_copy(srcs.at[w], srcv)
        pltpu.sync_copy(ones, onesv)
        plsc.subcore_barrier()

        def body(j, carry):
            pltpu.sync_copy(onesv, acc.at[srcv.at[j]], add=True)
            return carry

        lax.fori_loop(0, NCHUNK, body, 0)
        plsc.subcore_barrier()
        pltpu.sync_copy(acc.at[pl.ds(s * _ZROWS, _ZROWS)],
                        out.at[c, pl.ds(s * _ZROWS, _ZROWS)])

    return sc_scatter, sc_degree


def _dot(a, b):
    return jnp.dot(a, b, preferred_element_type=jnp.float32,
                   precision=lax.Precision.HIGHEST)


def _tca_body(v_ref, w_ref, deg_ref, out0_ref, u0_ref, dinv_ref):
    deg = deg_ref[0, :, 0:1] + deg_ref[1, :, 0:1]
    dinv = jnp.where(deg > 0, lax.rsqrt(jnp.maximum(deg, 1e-12)), 0.0)
    dinv_b = jnp.broadcast_to(dinv, (NB, D))
    x = v_ref[...]
    out0_ref[...] = _dot(x, w_ref[...])
    u0_ref[...] = dinv_b * x
    dinv_ref[...] = dinv_b


_tca1 = pl.pallas_call(
    _tca_body,
    grid=(GRID,),
    in_specs=[
        pl.BlockSpec((NB, D), lambda i: (i, 0)),
        pl.BlockSpec((D, D), lambda i: (0, 0)),
        pl.BlockSpec((2, NB, DEGW), lambda i: (0, i, 0)),
    ],
    out_specs=[
        pl.BlockSpec((NB, D), lambda i: (i, 0)),
        pl.BlockSpec((NB, D), lambda i: (i, 0)),
        pl.BlockSpec((NB, D), lambda i: (i, 0)),
    ],
    out_shape=[
        jax.ShapeDtypeStruct((N, D), jnp.float32),
        jax.ShapeDtypeStruct((NPAD, D), jnp.float32),
        jax.ShapeDtypeStruct((N, D), jnp.float32),
    ],
)


def _make_tcb(hh, do):
    di = hh * D

    def body(*refs):
        s_refs = refs[:hh]
        dinv_ref, out0_ref, w_ref = refs[hh:hh + 3]
        out1_ref = refs[hh + 3]
        u_refs = refs[hh + 4:]
        dinv = dinv_ref[...]
        txh = []
        for h in range(hh):
            sh = s_refs[h][...]
            t = -dinv * (sh[0] + sh[1])
            txh.append(t)
            u_refs[h][...] = dinv * t
        tx1 = jnp.concatenate(txh, axis=1) if hh > 1 else txh[0]
        out1_ref[...] = out0_ref[...] + _dot(tx1, w_ref[...])

    return pl.pallas_call(
        body,
        grid=(GRID,),
        in_specs=(
            [pl.BlockSpec((2, NB, D), lambda i: (0, i, 0)) for _ in range(hh)]
            + [
                pl.BlockSpec((NB, D), lambda i: (i, 0)),
                pl.BlockSpec((NB, do), lambda i: (i, 0)),
                pl.BlockSpec((di, do), lambda i: (0, 0)),
            ]
        ),
        out_specs=(
            [pl.BlockSpec((NB, do), lambda i: (i, 0))]
            + [pl.BlockSpec((NB, D), lambda i: (i, 0)) for _ in range(hh)]
        ),
        out_shape=(
            [jax.ShapeDtypeStruct((N, do), jnp.float32)]
            + [jax.ShapeDtypeStruct((NPAD, D), jnp.float32) for _ in range(hh)]
        ),
    )


def _make_tcc(hh, do, dn):
    di = hh * D
    ho = do // D

    def body(*refs):
        s_refs = refs[:hh]
        dinv_ref, out1_ref, x0_ref, w_ref, b_ref = refs[hh:hh + 5]
        k = hh + 5
        wn_ref = refs[k] if dn else None
        k += 1 if dn else 0
        h_ref = refs[k]
        u_refs = refs[k + 1:k + 1 + (ho if dn else 0)]
        o0_ref = refs[k + 1 + ho] if dn else None

        dinv = dinv_ref[...]
        x0 = x0_ref[...]
        txh = []
        for h in range(hh):
            sh = s_refs[h][...]
            txh.append(-2.0 * dinv * (sh[0] + sh[1]) - x0[:, h * D:(h + 1) * D])
        tx2 = jnp.concatenate(txh, axis=1) if hh > 1 else txh[0]
        hm = jax.nn.relu(out1_ref[...] + _dot(tx2, w_ref[...]) + b_ref[...])
        h_ref[...] = hm
        if dn:
            for h in range(ho):
                u_refs[h][...] = dinv * hm[:, h * D:(h + 1) * D]
            o0_ref[...] = _dot(hm, wn_ref[...])

    in_specs = (
        [pl.BlockSpec((2, NB, D), lambda i: (0, i, 0)) for _ in range(hh)]
        + [
            pl.BlockSpec((NB, D), lambda i: (i, 0)),
            pl.BlockSpec((NB, do), lambda i: (i, 0)),
            pl.BlockSpec((NB, di), lambda i: (i, 0)),
            pl.BlockSpec((di, do), lambda i: (0, 0)),
            pl.BlockSpec((1, do), lambda i: (0, 0)),
        ]
    )
    out_specs = [pl.BlockSpec((NB, do), lambda i: (i, 0))]
    out_shape = [jax.ShapeDtypeStruct((N, do), jnp.float32)]
    if dn:
        in_specs.append(pl.BlockSpec((do, dn), lambda i: (0, 0)))
        out_specs += [pl.BlockSpec((NB, D), lambda i: (i, 0)) for _ in range(ho)]
        out_shape += [jax.ShapeDtypeStruct((NPAD, D), jnp.float32) for _ in range(ho)]
        out_specs.append(pl.BlockSpec((NB, dn), lambda i: (i, 0)))
        out_shape.append(jax.ShapeDtypeStruct((N, dn), jnp.float32))

    return pl.pallas_call(body, grid=(GRID,), in_specs=in_specs,
                          out_specs=out_specs, out_shape=out_shape)


_tcb1 = _make_tcb(1, 128)
_tcc1 = _make_tcc(1, 128, 256)
_tcb2 = _make_tcb(1, 256)
_tcc2 = _make_tcc(1, 256, 512)
_tcb3 = _make_tcb(2, 512)
_tcc3 = _make_tcc(2, 512, 0)


def kernel(v, edge_index, W1, b1, W2, b2, W3, b3):
    e = edge_index.shape[1]
    pad = jnp.full((EPAD - e,), N, dtype=jnp.int32)
    srcs = jnp.concatenate([edge_index[0].astype(jnp.int32), pad]
                           ).reshape(NTILES, NCHUNK, CHUNK)
    dsts = jnp.concatenate([edge_index[1].astype(jnp.int32), pad]
                           ).reshape(NTILES, NCHUNK, CHUNK)
    zeros128 = jnp.zeros((NPAD, D), jnp.float32)
    zerosw = jnp.zeros((NPAD, DEGW), jnp.float32)
    onesw = jnp.ones((CHUNK, DEGW), jnp.float32)

    _sc_scatter, _sc_degree = _sc_kernels()
    deg = _sc_degree(srcs, zerosw, onesw)
    out0, u0, dinv = _tca1(v, W1[0], deg)

    s0 = _sc_scatter(u0, srcs, dsts, zeros128)
    out1, u1 = _tcb1(s0, dinv, out0, W1[1])
    s1 = _sc_scatter(u1, srcs, dsts, zeros128)
    h1, u0b, out0b = _tcc1(s1, dinv, out1, v, W1[2], b1.reshape(1, -1), W2[0])

    s0 = _sc_scatter(u0b, srcs, dsts, zeros128)
    out1, u1 = _tcb2(s0, dinv, out0b, W2[1])
    s1 = _sc_scatter(u1, srcs, dsts, zeros128)
    h2, u0c0, u0c1, out0c = _tcc2(s1, dinv, out1, h1, W2[2], b2.reshape(1, -1),
                                  W3[0])

    s0a = _sc_scatter(u0c0, srcs, dsts, zeros128)
    s0b = _sc_scatter(u0c1, srcs, dsts, zeros128)
    out1, u1a, u1b = _tcb3(s0a, s0b, dinv, out0c, W3[1])
    s1a = _sc_scatter(u1a, srcs, dsts, zeros128)
    s1b = _sc_scatter(u1b, srcs, dsts, zeros128)
    h3 = _tcc3(s1a, s1b, dinv, out1, h2, W3[2], b3.reshape(1, -1))
    return h3[0]
